# baseline (speedup 1.0000x reference)
import functools
import math

import jax
import jax.numpy as jnp
from jax import lax
from jax.experimental import pallas as pl
from jax.experimental.pallas import tpu as pltpu
from jax.experimental.pallas import tpu_sc as plsc


def _lag_tree(x, unit):
    while x.shape[0] > 1:
        h = x.shape[0] // 2
        x = x[:h] + jnp.roll(x[h:], h * unit, axis=-1)
    return x


def _corr_body(TM, qhi_ref, qlo_ref, khi_ref, klo_ref, out_ref, strip_ref):
    s = pl.program_id(1)
    ns = pl.num_programs(1)

    def mm(a_ref, b_ref):
        return jax.lax.dot_general(
            a_ref[0], b_ref[0], (((1,), (1,)), ((), ())),
            preferred_element_type=jnp.float32)

    prod = mm(qhi_ref, khi_ref) + mm(qhi_ref, klo_ref) + mm(qlo_ref, khi_ref)
    strip_ref[pl.ds(s, 1), :] = _lag_tree(prod, 1)

    @pl.when(s == ns - 1)
    def _reduce():
        out_ref[0] = jnp.roll(_lag_tree(strip_ref[:, :], TM), 1, axis=-1)


def _topk_body(K, L, HE, B, mv_ref, idx_ref, w_ref):
    mv = mv_ref[...]
    sel = jnp.sum(mv, axis=0, keepdims=True)
    iota = jax.lax.broadcasted_iota(jnp.int32, (1, L), 1)
    iota_k = jax.lax.broadcasted_iota(jnp.int32, (1, K), 1)
    idx_vec = jnp.zeros((1, K), jnp.int32)
    w_mat = jnp.zeros((B, K), jnp.float32)
    for i in range(K):
        m = jnp.max(sel)
        ii = jnp.min(jnp.where(sel == m, iota, L))
        idx_vec = jnp.where(iota_k == i, ii, idx_vec)
        wcol = jnp.sum(jnp.where(iota == ii, mv, 0.0), axis=1, keepdims=True)
        w_mat = jnp.where(iota_k == i, wcol, w_mat)
        sel = jnp.where(iota == ii, -jnp.inf, sel)
    idx_ref[...] = idx_vec
    w = w_mat * (1.0 / HE)
    w = jnp.exp(w - jnp.max(w, axis=1, keepdims=True))
    w_ref[...] = w / jnp.sum(w, axis=1, keepdims=True)


def _sc_agg_body(L, D, K, CH, rpw, wpb,
                 vflat_ref, idxb_ref, wb_ref, out_ref,
                 ibuf, wbuf, gbuf, obuf, sem):
    wid = lax.axis_index("s") * 2 + lax.axis_index("c")
    b = wid // wpb
    l0 = (wid % wpb) * rpw
    base = b * L
    pltpu.sync_copy(idxb_ref, ibuf)
    pltpu.sync_copy(wb_ref.at[pl.ds(b * K, K), :], wbuf)
    nvec = D // 16
    GR = 4

    def chunk_body(c, carry):
        row_l = l0 + c * CH + lax.iota(jnp.int32, 16)
        for g0 in range(0, K, GR):
            gsz = min(GR, K - g0)
            cps = []
            for j in range(gsz):
                li = row_l + ibuf[g0 + j, :]
                li = jnp.where(li >= L, li - L, li)
                cps.append(pltpu.async_copy(
                    vflat_ref.at[li + base], gbuf.at[j], sem))
            for cp in cps:
                cp.wait()

            def row_body(r, c2):
                def vec_body(v, c3):
                    sl = pl.ds(v * 16, 16)
                    acc = gbuf[0, r, sl] * wbuf[g0, :]
                    for j in range(1, gsz):
                        acc = acc + gbuf[j, r, sl] * wbuf[g0 + j, :]
                    if g0 == 0:
                        obuf[r, sl] = acc
                    else:
                        obuf[r, sl] = obuf[r, sl] + acc
                    return c3
                return lax.fori_loop(0, nvec, vec_body, c2)
            lax.fori_loop(0, CH, row_body, 0)
        pltpu.sync_copy(obuf, out_ref.at[pl.ds(base + l0 + c * CH, CH), :])
        return carry

    lax.fori_loop(0, rpw // CH, chunk_body, 0)


def kernel(queries, keys, values, attn_mask):
    B, L, H, E = queries.shape
    D = H * E
    K = int(1 * math.log(L))
    TM = TN = 256 if L % 256 == 0 else 64
    TB = 256 if L % 256 == 0 else 64
    nblk = L // TN

    qf = queries.reshape(B, L, D)
    kf = jnp.flip(keys.reshape(B, L, D), axis=1)
    qhi = qf.astype(jnp.bfloat16)
    qlo = (qf - qhi.astype(jnp.float32)).astype(jnp.bfloat16)
    khi = kf.astype(jnp.bfloat16)
    klo = (kf - khi.astype(jnp.float32)).astype(jnp.bfloat16)
    vflat = values.reshape(B * L, D)

    mv = pl.pallas_call(
        functools.partial(_corr_body, TM),
        grid=(B, L // TM),
        in_specs=[
            pl.BlockSpec((1, TM, D), lambda b, s: (b, s, 0)),
            pl.BlockSpec((1, TM, D), lambda b, s: (b, s, 0)),
            pl.BlockSpec((1, L, D), lambda b, s: (b, 0, 0)),
            pl.BlockSpec((1, L, D), lambda b, s: (b, 0, 0)),
        ],
        out_specs=pl.BlockSpec((1, 1, L), lambda b, s: (b, 0, 0)),
        out_shape=jax.ShapeDtypeStruct((B, 1, L), jnp.float32),
        scratch_shapes=[pltpu.VMEM((L // TM, L), jnp.float32)],
        compiler_params=pltpu.CompilerParams(
            dimension_semantics=("parallel", "arbitrary")),
    )(qhi, qlo, khi, klo)
    mv = mv.reshape(B, L)

    idx, w = pl.pallas_call(
        functools.partial(_topk_body, K, L, D, B),
        in_specs=[pl.BlockSpec((B, L), lambda: (0, 0))],
        out_specs=[
            pl.BlockSpec((1, K), lambda: (0, 0)),
            pl.BlockSpec((B, K), lambda: (0, 0)),
        ],
        out_shape=[
            jax.ShapeDtypeStruct((1, K), jnp.int32),
            jax.ShapeDtypeStruct((B, K), jnp.float32),
        ],
    )(mv)

    NW = 32
    rpw = (B * L) // NW
    wpb = L // rpw
    CH = 16
    idxb = jnp.broadcast_to(idx.reshape(K, 1), (K, 16)).astype(jnp.int32)
    wb = jnp.broadcast_to(w.reshape(B * K, 1), (B * K, 16))

    agg = functools.partial(
        pl.kernel,
        mesh=plsc.VectorSubcoreMesh(core_axis_name="c", subcore_axis_name="s"),
        out_type=jax.ShapeDtypeStruct((B * L, D), jnp.float32),
        scratch_types=[
            pltpu.VMEM((K, 16), jnp.int32),
            pltpu.VMEM((K, 16), jnp.float32),
            pltpu.VMEM((4, CH, D), jnp.float32),
            pltpu.VMEM((CH, D), jnp.float32),
            pltpu.SemaphoreType.DMA,
        ],
    )(functools.partial(_sc_agg_body, L, D, K, CH, rpw, wpb))
    out = agg(vflat, idxb, wb)

    return out.reshape(B, L, H, E)


# speedup vs baseline: 1.0062x; 1.0062x over previous
import functools
import math

import jax
import jax.numpy as jnp
from jax import lax
from jax.experimental import pallas as pl
from jax.experimental.pallas import tpu as pltpu
from jax.experimental.pallas import tpu_sc as plsc


def _lag_tree(x, unit):
    while x.shape[0] > 1:
        h = x.shape[0] // 2
        x = x[:h] + jnp.roll(x[h:], h * unit, axis=-1)
    return x


def _corr_body(TM, qhi_ref, qlo_ref, khi_ref, klo_ref, out_ref, strip_ref):
    s = pl.program_id(1)
    ns = pl.num_programs(1)

    def mm(a_ref, b_ref):
        return jax.lax.dot_general(
            a_ref[0], b_ref[0], (((1,), (1,)), ((), ())),
            preferred_element_type=jnp.float32)

    prod = mm(qhi_ref, khi_ref) + mm(qhi_ref, klo_ref) + mm(qlo_ref, khi_ref)
    strip_ref[pl.ds(s, 1), :] = _lag_tree(prod, 1)

    @pl.when(s == ns - 1)
    def _reduce():
        out_ref[0] = jnp.roll(_lag_tree(strip_ref[:, :], TM), 1, axis=-1)


def _topk_body(K, L, HE, B, mv_ref, idx_ref, w_ref):
    mv = mv_ref[...]
    sel = jnp.sum(mv, axis=0, keepdims=True)
    iota = jax.lax.broadcasted_iota(jnp.int32, (1, L), 1)
    iota_k = jax.lax.broadcasted_iota(jnp.int32, (1, K), 1)
    idx_vec = jnp.zeros((1, K), jnp.int32)
    w_mat = jnp.zeros((B, K), jnp.float32)
    for i in range(K):
        m = jnp.max(sel)
        ii = jnp.min(jnp.where(sel == m, iota, L))
        idx_vec = jnp.where(iota_k == i, ii, idx_vec)
        wcol = jnp.sum(jnp.where(iota == ii, mv, 0.0), axis=1, keepdims=True)
        w_mat = jnp.where(iota_k == i, wcol, w_mat)
        sel = jnp.where(iota == ii, -jnp.inf, sel)
    idx_ref[...] = idx_vec
    w = w_mat * (1.0 / HE)
    w = jnp.exp(w - jnp.max(w, axis=1, keepdims=True))
    w_ref[...] = w / jnp.sum(w, axis=1, keepdims=True)


def _sc_agg_body(L, D, K, CH, rpw, wpb,
                 vflat_ref, idxb_ref, wb_ref, out_ref,
                 ibuf, wbuf, gbuf, obuf, sem):
    wid = lax.axis_index("s") * 2 + lax.axis_index("c")
    b = wid // wpb
    l0 = (wid % wpb) * rpw
    base = b * L
    pltpu.sync_copy(idxb_ref, ibuf)
    pltpu.sync_copy(wb_ref.at[pl.ds(b * K, K), :], wbuf)
    nvec = D // 16
    GR = 4

    def chunk_body(c, carry):
        row_l = l0 + c * CH + lax.iota(jnp.int32, 16)
        for g0 in range(0, K, GR):
            gsz = min(GR, K - g0)
            cps = []
            for j in range(gsz):
                li = row_l + ibuf[g0 + j, :]
                li = jnp.where(li >= L, li - L, li)
                cps.append(pltpu.async_copy(
                    vflat_ref.at[li + base], gbuf.at[j], sem))
            for cp in cps:
                cp.wait()

            def vec_body(v, c2):
                sl = pl.ds(v * 16, 16)
                for r in range(CH):
                    acc = gbuf[0, r, sl] * wbuf[g0, :]
                    for j in range(1, gsz):
                        acc = acc + gbuf[j, r, sl] * wbuf[g0 + j, :]
                    if g0 == 0:
                        obuf[r, sl] = acc
                    else:
                        obuf[r, sl] = obuf[r, sl] + acc
                return c2
            lax.fori_loop(0, nvec, vec_body, 0)
        pltpu.sync_copy(obuf, out_ref.at[pl.ds(base + l0 + c * CH, CH), :])
        return carry

    lax.fori_loop(0, rpw // CH, chunk_body, 0)


def kernel(queries, keys, values, attn_mask):
    B, L, H, E = queries.shape
    D = H * E
    K = int(1 * math.log(L))
    TM = TN = 256 if L % 256 == 0 else 64
    TB = 256 if L % 256 == 0 else 64
    nblk = L // TN

    qf = queries.reshape(B, L, D)
    kf = jnp.flip(keys.reshape(B, L, D), axis=1)
    qhi = qf.astype(jnp.bfloat16)
    qlo = (qf - qhi.astype(jnp.float32)).astype(jnp.bfloat16)
    khi = kf.astype(jnp.bfloat16)
    klo = (kf - khi.astype(jnp.float32)).astype(jnp.bfloat16)
    vflat = values.reshape(B * L, D)

    mv = pl.pallas_call(
        functools.partial(_corr_body, TM),
        grid=(B, L // TM),
        in_specs=[
            pl.BlockSpec((1, TM, D), lambda b, s: (b, s, 0)),
            pl.BlockSpec((1, TM, D), lambda b, s: (b, s, 0)),
            pl.BlockSpec((1, L, D), lambda b, s: (b, 0, 0)),
            pl.BlockSpec((1, L, D), lambda b, s: (b, 0, 0)),
        ],
        out_specs=pl.BlockSpec((1, 1, L), lambda b, s: (b, 0, 0)),
        out_shape=jax.ShapeDtypeStruct((B, 1, L), jnp.float32),
        scratch_shapes=[pltpu.VMEM((L // TM, L), jnp.float32)],
        compiler_params=pltpu.CompilerParams(
            dimension_semantics=("parallel", "arbitrary")),
    )(qhi, qlo, khi, klo)
    mv = mv.reshape(B, L)

    idx, w = pl.pallas_call(
        functools.partial(_topk_body, K, L, D, B),
        in_specs=[pl.BlockSpec((B, L), lambda: (0, 0))],
        out_specs=[
            pl.BlockSpec((1, K), lambda: (0, 0)),
            pl.BlockSpec((B, K), lambda: (0, 0)),
        ],
        out_shape=[
            jax.ShapeDtypeStruct((1, K), jnp.int32),
            jax.ShapeDtypeStruct((B, K), jnp.float32),
        ],
    )(mv)

    NW = 32
    rpw = (B * L) // NW
    wpb = L // rpw
    CH = 16
    idxb = jnp.broadcast_to(idx.reshape(K, 1), (K, 16)).astype(jnp.int32)
    wb = jnp.broadcast_to(w.reshape(B * K, 1), (B * K, 16))

    agg = functools.partial(
        pl.kernel,
        mesh=plsc.VectorSubcoreMesh(core_axis_name="c", subcore_axis_name="s"),
        out_type=jax.ShapeDtypeStruct((B * L, D), jnp.float32),
        scratch_types=[
            pltpu.VMEM((K, 16), jnp.int32),
            pltpu.VMEM((K, 16), jnp.float32),
            pltpu.VMEM((4, CH, D), jnp.float32),
            pltpu.VMEM((CH, D), jnp.float32),
            pltpu.SemaphoreType.DMA,
        ],
    )(functools.partial(_sc_agg_body, L, D, K, CH, rpw, wpb))
    out = agg(vflat, idxb, wb)

    return out.reshape(B, L, H, E)
